# baseline (speedup 1.0000x reference)
import functools

import jax
import jax.numpy as jnp
from jax import lax
from jax.experimental import pallas as pl
from jax.experimental.pallas import tpu as pltpu
from jax.experimental.pallas import tpu_sc as plsc

NC = 2
NS = 16
NW = NC * NS
CH = 128
NBUF = 4
TB = 2


@functools.partial(jax.jit, static_argnames=("v", "d"))
def _sc_transpose(params_t, tail_lin, v, d):
    n_blk = v // CH
    n_tail = v - n_blk * CH
    mesh = plsc.VectorSubcoreMesh(core_axis_name="c", subcore_axis_name="s")

    @functools.partial(
        pl.kernel,
        mesh=mesh,
        out_type=jax.ShapeDtypeStruct((v * d,), jnp.float32),
        scratch_types=[
            pltpu.VMEM((TB, d // 8, 8, CH + 1), jnp.float32),
            pltpu.VMEM((TB, d * CH), jnp.float32),
        ]
        + [pltpu.SemaphoreType.DMA] * (2 * TB),
        compiler_params=pltpu.CompilerParams(
            use_tc_tiling_on_sc=True, needs_layout_passes=False
        ),
    )
    def k(tin, tail, tout, blk_v, dense_v, *sems):
        in_sems = sems[:TB]
        out_sems = sems[TB:]
        wid = lax.axis_index("s") * NC + lax.axis_index("c")
        per = n_blk // NW
        extra = n_blk % NW
        start = wid * per + jnp.minimum(wid, extra)
        count = per + jnp.where(wid < extra, 1, 0)

        if n_tail:

            @pl.when(wid == NW - 1)
            def _():
                pltpu.sync_copy(tail, tout.at[pl.ds(n_blk * CH * d, n_tail * d)])

        def row0(b):
            return (start + b) * CH

        def get_copies(b, s):
            return [
                pltpu.make_async_copy(
                    tin.at[pl.ds(8 * ct, 8), pl.ds(row0(b), CH)],
                    blk_v.at[s, ct, :, pl.ds(0, CH)],
                    in_sems[s],
                )
                for ct in range(d // 8)
            ]

        def put(b, s):
            return pltpu.make_async_copy(
                dense_v.at[s], tout.at[pl.ds(row0(b) * d, CH * d)], out_sems[s]
            )

        lanes = lax.iota(jnp.int32, 16)
        ct_g = [2 * g + lanes // 8 for g in range(d // 16)]
        cs_g = [lanes % 8 for _ in range(d // 16)]

        for s in range(TB):
            for c in get_copies(s, s):
                c.start()

        def body(b, carry):
            s_dyn = b % TB
            for s in range(TB):

                @pl.when(s_dyn == s)
                def _():
                    for c in get_copies(b, s):
                        c.wait()

                    @plsc.parallel_loop(0, CH, unroll=8)
                    def _(i):
                        row = jnp.full((16,), i, jnp.int32)
                        for g in range(d // 16):
                            vec = plsc.load_gather(
                                blk_v.at[s], [ct_g[g], cs_g[g], row]
                            )
                            dense_v[s, pl.ds(i * d + g * 16, 16)] = vec

                    put(b, s).start()

                    @pl.when(b + TB < count)
                    def _():
                        put(b, s).wait()
                        for c in get_copies(b + TB, s):
                            c.start()

            return carry

        lax.fori_loop(0, count, body, 0)
        for s in range(TB):

            @pl.when(s < count)
            def _():
                put(jnp.maximum(count - TB, 0) + s, s).wait()

    return k(params_t, tail_lin)


@functools.partial(jax.jit, static_argnames=("n_chunks", "d"))
def _sc_gather(idx3, table, n_chunks, d):
    btot = NW * n_chunks * CH
    mesh = plsc.VectorSubcoreMesh(core_axis_name="c", subcore_axis_name="s")

    @functools.partial(
        pl.kernel,
        mesh=mesh,
        out_type=jax.ShapeDtypeStruct((btot, 2 * d), jnp.float32),
        scratch_types=[
            pltpu.VMEM((n_chunks, CH), jnp.int32),
            pltpu.VMEM((NBUF, CH, d), jnp.float32),
        ]
        + [pltpu.SemaphoreType.DMA] * (2 * NBUF),
        compiler_params=pltpu.CompilerParams(use_tc_tiling_on_sc=False),
    )
    def k(table_hbm, idx_hbm, out_hbm, idx_v, rows_v, *sems):
        in_sems = sems[:NBUF]
        out_sems = sems[NBUF:]
        wid = lax.axis_index("s") * NC + lax.axis_index("c")
        base = wid * (n_chunks * CH)

        pltpu.sync_copy(idx_hbm.at[wid], idx_v)

        def gather(j, b):
            return pltpu.make_async_copy(
                table_hbm.at[idx_v.at[j]], rows_v.at[b], in_sems[b]
            )

        def put(j, b):
            return pltpu.make_async_copy(
                rows_v.at[b],
                out_hbm.at[pl.ds(base + j * CH, CH), pl.ds(0, d)],
                out_sems[b],
            )

        for b in range(NBUF):
            gather(b, b).start()

        def body(g, carry):
            for b in range(NBUF):
                j = g * NBUF + b
                gather(j, b).wait()
                put(j, b).start()
                put(j, b).wait()
                gather(j + NBUF, b).start()
            return carry

        lax.fori_loop(0, (n_chunks - NBUF) // NBUF, body, 0)

        for b in range(NBUF):
            j = (n_chunks - NBUF) + b
            gather(j, b).wait()
            put(j, b).start()
        for b in range(NBUF):
            j = (n_chunks - NBUF) + b
            put(j, b).wait()

    return k(table, idx3)


def kernel(token_index, params):
    b, t = token_index.shape
    v, d = params.shape
    n_full = (v // CH) * CH
    tail_lin = params[n_full:].reshape(-1)
    dense = _sc_transpose(params.T, tail_lin, v, d)
    table = dense.reshape(v, d)
    flat = token_index.reshape(-1).astype(jnp.int32)
    n_chunks = flat.shape[0] // (NW * CH)
    idx3 = flat.reshape(NW, n_chunks, CH)
    out = _sc_gather(idx3, table, n_chunks, d)
    return out[:, :d].reshape(b, t, d)


# speedup vs baseline: 1.3144x; 1.3144x over previous
import functools

import jax
import jax.numpy as jnp
from jax import lax
from jax.experimental import pallas as pl
from jax.experimental.pallas import tpu as pltpu
from jax.experimental.pallas import tpu_sc as plsc

NC = 2
NS = 16
NW = NC * NS
CH = 128
NBUF = 4


@functools.partial(jax.jit, static_argnames=("n_chunks", "d"))
def _sc_gather(idx3, table, n_chunks, d):
    btot = NW * n_chunks * CH
    mesh = plsc.VectorSubcoreMesh(core_axis_name="c", subcore_axis_name="s")

    @functools.partial(
        pl.kernel,
        mesh=mesh,
        out_type=jax.ShapeDtypeStruct((btot, 2 * d), jnp.float32),
        scratch_types=[
            pltpu.VMEM((n_chunks, CH), jnp.int32),
            pltpu.VMEM((NBUF, CH, d), jnp.float32),
        ]
        + [pltpu.SemaphoreType.DMA] * (2 * NBUF),
        compiler_params=pltpu.CompilerParams(use_tc_tiling_on_sc=False),
    )
    def k(table_hbm, idx_hbm, out_hbm, idx_v, rows_v, *sems):
        in_sems = sems[:NBUF]
        out_sems = sems[NBUF:]
        wid = lax.axis_index("s") * NC + lax.axis_index("c")
        base = wid * (n_chunks * CH)

        pltpu.sync_copy(idx_hbm.at[wid], idx_v)

        def gather(j, b):
            return pltpu.make_async_copy(
                table_hbm.at[idx_v.at[j]], rows_v.at[b], in_sems[b]
            )

        def put(j, b):
            return pltpu.make_async_copy(
                rows_v.at[b],
                out_hbm.at[pl.ds(base + j * CH, CH), pl.ds(0, d)],
                out_sems[b],
            )

        for b in range(NBUF):
            gather(b, b).start()

        def body(g, carry):
            for b in range(NBUF):
                j = g * NBUF + b
                gather(j, b).wait()
                put(j, b).start()
                put(j, b).wait()
                gather(j + NBUF, b).start()
            return carry

        lax.fori_loop(0, (n_chunks - NBUF) // NBUF, body, 0)

        for b in range(NBUF):
            j = (n_chunks - NBUF) + b
            gather(j, b).wait()
            put(j, b).start()
        for b in range(NBUF):
            j = (n_chunks - NBUF) + b
            put(j, b).wait()

    return k(table, idx3)


def kernel(token_index, params):
    b, t = token_index.shape
    v, d = params.shape
    ph = lax.optimization_barrier(params.reshape(v // 2, 2 * d))
    table = ph.reshape(v, d)
    flat = token_index.reshape(-1).astype(jnp.int32)
    n_chunks = flat.shape[0] // (NW * CH)
    idx3 = flat.reshape(NW, n_chunks, CH)
    out = _sc_gather(idx3, table, n_chunks, d)
    return out[:, :d].reshape(b, t, d)
